# Initial kernel scaffold; baseline (speedup 1.0000x reference)
import jax
import jax.numpy as jnp
from jax.experimental import pallas as pl


def kernel(x, edge_index, edge_attr, batch, params):
    raise NotImplementedError("write your pallas kernel here")


# speedup vs baseline: 4.8243x; 4.8243x over previous
import functools

import jax
import jax.numpy as jnp
from jax import lax
from jax.experimental import pallas as pl
from jax.experimental.pallas import tpu as pltpu
from jax.experimental.pallas import tpu_sc as plsc

_N = 10000
_E = 320000
_DIM = 128
_DE = 16
_G = 64
_LAT = 64

_NC = 2
_NS = 16
_NW = _NC * _NS
_EPW = _E // _NW
_CH = 128
_NFULL = _EPW // _CH
_REM = _EPW - _NFULL * _CH
_NP = 10240
_RPT = _NP // _NS

_ROWS = 1000
_NBLK = _N // _ROWS


def _make_sc_agg(depth, gather):
    mesh = plsc.VectorSubcoreMesh(
        core_axis_name="c", subcore_axis_name="s",
        num_cores=_NC, num_subcores=_NS)

    @functools.partial(
        pl.kernel,
        out_type=jax.ShapeDtypeStruct((_NC, _NP, depth), jnp.float32),
        mesh=mesh,
        scratch_types=[
            pltpu.VMEM((_CH,), jnp.int32),
            pltpu.VMEM((_CH,), jnp.int32),
            pltpu.VMEM((_CH, depth), jnp.float32),
            pltpu.VMEM((_REM,), jnp.int32),
            pltpu.VMEM((_REM,), jnp.int32),
            pltpu.VMEM((_REM, depth), jnp.float32),
            pltpu.VMEM_SHARED((_NP, depth), jnp.float32),
            pltpu.SemaphoreType.DMA,
        ],
    )
    def sc_agg(rows_hbm, src_hbm, dst_hbm, zeros_hbm, out_hbm,
               src_v, dst_v, buf_v, srcr_v, dstr_v, bufr_v, acc_sh, sem):
        cid = lax.axis_index("c")
        sid = lax.axis_index("s")
        zoff = pl.multiple_of(sid * _RPT, 8)
        pltpu.sync_copy(zeros_hbm.at[pl.ds(zoff, _RPT)],
                        acc_sh.at[pl.ds(zoff, _RPT)])
        plsc.subcore_barrier()

        base = (cid * _NS + sid) * _EPW

        def body(c, carry):
            off = pl.multiple_of(base + c * _CH, 8)
            pltpu.sync_copy(dst_hbm.at[pl.ds(off, _CH)], dst_v)
            if gather:
                pltpu.sync_copy(src_hbm.at[pl.ds(off, _CH)], src_v)
                pltpu.async_copy(rows_hbm.at[src_v], buf_v, sem).wait()
            else:
                pltpu.sync_copy(rows_hbm.at[pl.ds(off, _CH)], buf_v)
            pltpu.sync_copy(buf_v, acc_sh.at[dst_v], add=True)
            return carry

        lax.fori_loop(0, _NFULL, body, 0)

        offr = pl.multiple_of(base + _NFULL * _CH, 8)
        pltpu.sync_copy(dst_hbm.at[pl.ds(offr, _REM)], dstr_v)
        if gather:
            pltpu.sync_copy(src_hbm.at[pl.ds(offr, _REM)], srcr_v)
            pltpu.async_copy(rows_hbm.at[srcr_v], bufr_v, sem).wait()
        else:
            pltpu.sync_copy(rows_hbm.at[pl.ds(offr, _REM)], bufr_v)
        pltpu.sync_copy(bufr_v, acc_sh.at[dstr_v], add=True)

        plsc.subcore_barrier()
        pltpu.sync_copy(acc_sh.at[pl.ds(zoff, _RPT)],
                        out_hbm.at[cid, pl.ds(zoff, _RPT)])

    return sc_agg


@functools.lru_cache(maxsize=None)
def _sc_agg(depth, gather):
    return _make_sc_agg(depth, gather)


def _sc_agg_h(rows, src, dst, zeros):
    return _sc_agg(_DIM, True)(rows, src, dst, zeros)


def _sc_agg_ea(rows, src, dst, zeros):
    return _sc_agg(_DIM, False)(rows, src, dst, zeros)


def _layer_body(h_ref, hagg_ref, eagg_ref, ws_ref, wn_ref, we_ref, b_ref,
                o_ref, *, act):
    f32 = jnp.float32
    acc = jnp.dot(h_ref[...], ws_ref[...], preferred_element_type=f32)
    acc += jnp.dot(hagg_ref[0] + hagg_ref[1], wn_ref[...],
                   preferred_element_type=f32)
    acc += jnp.dot(eagg_ref[0] + eagg_ref[1], we_ref[...],
                   preferred_element_type=f32)
    acc += b_ref[...]
    if act:
        acc = jnp.maximum(acc, 0.0)
    o_ref[...] = acc


def _layer_call(h, hagg, eagg, ws, wn, we, b, act):
    return pl.pallas_call(
        functools.partial(_layer_body, act=act),
        grid=(_NBLK,),
        in_specs=[
            pl.BlockSpec((_ROWS, _DIM), lambda i: (i, 0)),
            pl.BlockSpec((_NC, _ROWS, _DIM), lambda i: (0, i, 0)),
            pl.BlockSpec((_NC, _ROWS, _DIM), lambda i: (0, i, 0)),
            pl.BlockSpec((_DIM, _DIM), lambda i: (0, 0)),
            pl.BlockSpec((_DIM, _DIM), lambda i: (0, 0)),
            pl.BlockSpec((_DIM, _DIM), lambda i: (0, 0)),
            pl.BlockSpec((1, _DIM), lambda i: (0, 0)),
        ],
        out_specs=pl.BlockSpec((_ROWS, _DIM), lambda i: (i, 0)),
        out_shape=jax.ShapeDtypeStruct((_N, _DIM), jnp.float32),
    )(h, hagg, eagg, ws, wn, we, b.reshape(1, _DIM))


def _onehot(batch_ref):
    bvals = batch_ref[0, 0, :]
    return (bvals[:, None] == lax.broadcasted_iota(
        jnp.int32, (1, _G), 1)).astype(jnp.float32)


def _readout_body(h_ref, batch_ref, sums_ref, cnt_ref):
    i = pl.program_id(0)
    oh = _onehot(batch_ref)
    s = lax.dot_general(oh, h_ref[...], (((0,), (0,)), ((), ())),
                        preferred_element_type=jnp.float32)
    c = jnp.broadcast_to(jnp.sum(oh, axis=0)[:, None], (_G, _DIM))

    @pl.when(i == 0)
    def _():
        sums_ref[...] = s
        cnt_ref[...] = c

    @pl.when(i > 0)
    def _():
        sums_ref[...] += s
        cnt_ref[...] += c


def _readout_call(h, batch3):
    return pl.pallas_call(
        _readout_body,
        grid=(_NBLK,),
        in_specs=[
            pl.BlockSpec((_ROWS, _DIM), lambda i: (i, 0)),
            pl.BlockSpec((1, 1, _ROWS), lambda i: (i, 0, 0)),
        ],
        out_specs=[
            pl.BlockSpec((_G, _DIM), lambda i: (0, 0)),
            pl.BlockSpec((_G, _DIM), lambda i: (0, 0)),
        ],
        out_shape=[
            jax.ShapeDtypeStruct((_G, _DIM), jnp.float32),
            jax.ShapeDtypeStruct((_G, _DIM), jnp.float32),
        ],
    )(h, batch3)


def _decode_body(sums_ref, cnt_ref, eps_ref, batch_ref, w1_ref, b1_ref,
                 w2_ref, b2_ref, recon_ref, mu_ref, logvar_ref):
    f32 = jnp.float32
    g = sums_ref[...] / jnp.maximum(cnt_ref[...], 1.0)
    mu = g[:, :_LAT]
    logvar = g[:, _LAT:]
    mu_ref[...] = mu
    logvar_ref[...] = logvar
    z = mu + eps_ref[...] * jnp.exp(0.5 * logvar)
    oh = _onehot(batch_ref)
    z_exp = jnp.dot(oh, z, preferred_element_type=f32)
    hmid = jnp.maximum(
        jnp.dot(z_exp, w1_ref[...], preferred_element_type=f32) + b1_ref[...],
        0.0)
    recon_ref[...] = (
        jnp.dot(hmid, w2_ref[...], preferred_element_type=f32) + b2_ref[...])


def _decode_call(sums, cnt, eps, batch3, w1, b1, w2, b2):
    return pl.pallas_call(
        _decode_body,
        grid=(_NBLK,),
        in_specs=[
            pl.BlockSpec((_G, _DIM), lambda i: (0, 0)),
            pl.BlockSpec((_G, _DIM), lambda i: (0, 0)),
            pl.BlockSpec((_G, _LAT), lambda i: (0, 0)),
            pl.BlockSpec((1, 1, _ROWS), lambda i: (i, 0, 0)),
            pl.BlockSpec((_LAT, _DIM), lambda i: (0, 0)),
            pl.BlockSpec((1, _DIM), lambda i: (0, 0)),
            pl.BlockSpec((_DIM, _DIM), lambda i: (0, 0)),
            pl.BlockSpec((1, _DIM), lambda i: (0, 0)),
        ],
        out_specs=[
            pl.BlockSpec((_ROWS, _DIM), lambda i: (i, 0)),
            pl.BlockSpec((_G, _LAT), lambda i: (0, 0)),
            pl.BlockSpec((_G, _LAT), lambda i: (0, 0)),
        ],
        out_shape=[
            jax.ShapeDtypeStruct((_N, _DIM), jnp.float32),
            jax.ShapeDtypeStruct((_G, _LAT), jnp.float32),
            jax.ShapeDtypeStruct((_G, _LAT), jnp.float32),
        ],
    )(sums, cnt, eps, batch3, w1, b1.reshape(1, _DIM), w2, b2.reshape(1, _DIM))


def kernel(x, edge_index, edge_attr, batch, params):
    src = edge_index[0]
    dst = edge_index[1]
    eps = jax.random.normal(jax.random.key(42), (_G, _LAT), jnp.float32)
    zeros_h = jnp.zeros((_NP, _DIM), jnp.float32)
    batch3 = batch.reshape(_NBLK, 1, _ROWS)

    ea_pad = jnp.pad(edge_attr, ((0, 0), (0, _DIM - _DE)))
    eagg = _sc_agg_ea(ea_pad, dst, dst, zeros_h)
    h = x
    for l in range(3):
        hagg = _sc_agg_h(h, src, dst, zeros_h)
        h = _layer_call(h, hagg, eagg, params['W_self'][l],
                        params['W_nbr'][l], jnp.pad(params['W_edge'][l], ((0, _DIM - _DE), (0, 0))),
                        params['b'][l], act=(l < 2))
    sums, cnt = _readout_call(h, batch3)
    recon, mu, logvar = _decode_call(
        sums, cnt, eps, batch3, params['mlp_W1'], params['mlp_b1'],
        params['mlp_W2'], params['mlp_b2'])
    return (recon, mu, logvar)


# speedup vs baseline: 7.2496x; 1.5027x over previous
import functools

import jax
import jax.numpy as jnp
from jax import lax
from jax.experimental import pallas as pl
from jax.experimental.pallas import tpu as pltpu
from jax.experimental.pallas import tpu_sc as plsc

_N = 10000
_E = 320000
_DIM = 128
_DE = 16
_G = 64
_LAT = 64

_NC = 2
_NS = 16
_NW = _NC * _NS
_EPW = _E // _NW
_CH = 128
_NFULL = _EPW // _CH
_REM = _EPW - _NFULL * _CH
_NP = 10240
_RPT = _NP // _NS

_ROWS = 1000
_NBLK = _N // _ROWS


def _make_sc_agg(depth, gather):
    mesh = plsc.VectorSubcoreMesh(
        core_axis_name="c", subcore_axis_name="s",
        num_cores=_NC, num_subcores=_NS)

    npair = _NFULL // 2

    @functools.partial(
        pl.kernel,
        out_type=jax.ShapeDtypeStruct((_NC, _NP, depth), jnp.float32),
        mesh=mesh,
        scratch_types=[
            pltpu.VMEM((_CH,), jnp.int32),
            pltpu.VMEM((_CH,), jnp.int32),
            pltpu.VMEM((_CH, depth), jnp.float32),
            pltpu.VMEM((_CH,), jnp.int32),
            pltpu.VMEM((_CH,), jnp.int32),
            pltpu.VMEM((_CH, depth), jnp.float32),
            pltpu.VMEM((_REM,), jnp.int32),
            pltpu.VMEM((_REM,), jnp.int32),
            pltpu.VMEM((_REM, depth), jnp.float32),
            pltpu.VMEM_SHARED((_NP, depth), jnp.float32),
            pltpu.SemaphoreType.DMA,
            pltpu.SemaphoreType.DMA,
        ],
    )
    def sc_agg(rows_hbm, src_hbm, dst_hbm, zeros_hbm, out_hbm,
               src0, dst0, buf0, src1, dst1, buf1,
               srcr_v, dstr_v, bufr_v, acc_sh, sem0, sem1):
        cid = lax.axis_index("c")
        sid = lax.axis_index("s")
        zoff = pl.multiple_of(sid * _RPT, 8)
        pltpu.sync_copy(zeros_hbm.at[pl.ds(zoff, _RPT)],
                        acc_sh.at[pl.ds(zoff, _RPT)])
        plsc.subcore_barrier()

        base = (cid * _NS + sid) * _EPW

        def load_and_start(c, srcv, dstv, bufv, semv):
            off = pl.multiple_of(base + c * _CH, 8)
            pltpu.sync_copy(dst_hbm.at[pl.ds(off, _CH)], dstv)
            if gather:
                pltpu.sync_copy(src_hbm.at[pl.ds(off, _CH)], srcv)
                pltpu.async_copy(rows_hbm.at[srcv], bufv, semv)
            else:
                pltpu.async_copy(rows_hbm.at[pl.ds(off, _CH)], bufv, semv)

        def wait_and_scatter(bufv, dstv, semv):
            pltpu.make_async_copy(
                rows_hbm.at[pl.ds(0, _CH)], bufv, semv).wait()
            pltpu.sync_copy(bufv, acc_sh.at[dstv], add=True)

        load_and_start(0, src0, dst0, buf0, sem0)

        def body(i, carry):
            load_and_start(2 * i + 1, src1, dst1, buf1, sem1)
            wait_and_scatter(buf0, dst0, sem0)

            @pl.when(i < npair - 1)
            def _():
                load_and_start(2 * i + 2, src0, dst0, buf0, sem0)

            wait_and_scatter(buf1, dst1, sem1)
            return carry

        lax.fori_loop(0, npair, body, 0)

        offr = pl.multiple_of(base + _NFULL * _CH, 8)
        pltpu.sync_copy(dst_hbm.at[pl.ds(offr, _REM)], dstr_v)
        if gather:
            pltpu.sync_copy(src_hbm.at[pl.ds(offr, _REM)], srcr_v)
            pltpu.async_copy(rows_hbm.at[srcr_v], bufr_v, sem0).wait()
        else:
            pltpu.sync_copy(rows_hbm.at[pl.ds(offr, _REM)], bufr_v)
        pltpu.sync_copy(bufr_v, acc_sh.at[dstr_v], add=True)

        plsc.subcore_barrier()
        pltpu.sync_copy(acc_sh.at[pl.ds(zoff, _RPT)],
                        out_hbm.at[cid, pl.ds(zoff, _RPT)])

    return sc_agg


@functools.lru_cache(maxsize=None)
def _sc_agg(depth, gather):
    return _make_sc_agg(depth, gather)


def _sc_agg_h(rows, src, dst, zeros):
    return _sc_agg(_DIM, True)(rows, src, dst, zeros)


def _sc_agg_ea(rows, src, dst, zeros):
    return _sc_agg(_DIM, False)(rows, src, dst, zeros)


def _layer_body(h_ref, hagg_ref, eagg_ref, ws_ref, wn_ref, we_ref, b_ref,
                o_ref, *, act):
    f32 = jnp.float32
    acc = jnp.dot(h_ref[...], ws_ref[...], preferred_element_type=f32)
    acc += jnp.dot(hagg_ref[0] + hagg_ref[1], wn_ref[...],
                   preferred_element_type=f32)
    acc += jnp.dot(eagg_ref[0] + eagg_ref[1], we_ref[...],
                   preferred_element_type=f32)
    acc += b_ref[...]
    if act:
        acc = jnp.maximum(acc, 0.0)
    o_ref[...] = acc


def _layer_call(h, hagg, eagg, ws, wn, we, b, act):
    return pl.pallas_call(
        functools.partial(_layer_body, act=act),
        grid=(_NBLK,),
        in_specs=[
            pl.BlockSpec((_ROWS, _DIM), lambda i: (i, 0)),
            pl.BlockSpec((_NC, _ROWS, _DIM), lambda i: (0, i, 0)),
            pl.BlockSpec((_NC, _ROWS, _DIM), lambda i: (0, i, 0)),
            pl.BlockSpec((_DIM, _DIM), lambda i: (0, 0)),
            pl.BlockSpec((_DIM, _DIM), lambda i: (0, 0)),
            pl.BlockSpec((_DIM, _DIM), lambda i: (0, 0)),
            pl.BlockSpec((1, _DIM), lambda i: (0, 0)),
        ],
        out_specs=pl.BlockSpec((_ROWS, _DIM), lambda i: (i, 0)),
        out_shape=jax.ShapeDtypeStruct((_N, _DIM), jnp.float32),
    )(h, hagg, eagg, ws, wn, we, b.reshape(1, _DIM))


def _onehot(batch_ref):
    bvals = batch_ref[0, 0, :]
    return (bvals[:, None] == lax.broadcasted_iota(
        jnp.int32, (1, _G), 1)).astype(jnp.float32)


def _readout_body(h_ref, batch_ref, sums_ref, cnt_ref):
    i = pl.program_id(0)
    oh = _onehot(batch_ref)
    s = lax.dot_general(oh, h_ref[...], (((0,), (0,)), ((), ())),
                        preferred_element_type=jnp.float32)
    c = jnp.broadcast_to(jnp.sum(oh, axis=0)[:, None], (_G, _DIM))

    @pl.when(i == 0)
    def _():
        sums_ref[...] = s
        cnt_ref[...] = c

    @pl.when(i > 0)
    def _():
        sums_ref[...] += s
        cnt_ref[...] += c


def _readout_call(h, batch3):
    return pl.pallas_call(
        _readout_body,
        grid=(_NBLK,),
        in_specs=[
            pl.BlockSpec((_ROWS, _DIM), lambda i: (i, 0)),
            pl.BlockSpec((1, 1, _ROWS), lambda i: (i, 0, 0)),
        ],
        out_specs=[
            pl.BlockSpec((_G, _DIM), lambda i: (0, 0)),
            pl.BlockSpec((_G, _DIM), lambda i: (0, 0)),
        ],
        out_shape=[
            jax.ShapeDtypeStruct((_G, _DIM), jnp.float32),
            jax.ShapeDtypeStruct((_G, _DIM), jnp.float32),
        ],
    )(h, batch3)


def _decode_body(sums_ref, cnt_ref, eps_ref, batch_ref, w1_ref, b1_ref,
                 w2_ref, b2_ref, recon_ref, mu_ref, logvar_ref):
    f32 = jnp.float32
    g = sums_ref[...] / jnp.maximum(cnt_ref[...], 1.0)
    mu = g[:, :_LAT]
    logvar = g[:, _LAT:]
    mu_ref[...] = mu
    logvar_ref[...] = logvar
    z = mu + eps_ref[...] * jnp.exp(0.5 * logvar)
    oh = _onehot(batch_ref)
    z_exp = jnp.dot(oh, z, preferred_element_type=f32)
    hmid = jnp.maximum(
        jnp.dot(z_exp, w1_ref[...], preferred_element_type=f32) + b1_ref[...],
        0.0)
    recon_ref[...] = (
        jnp.dot(hmid, w2_ref[...], preferred_element_type=f32) + b2_ref[...])


def _decode_call(sums, cnt, eps, batch3, w1, b1, w2, b2):
    return pl.pallas_call(
        _decode_body,
        grid=(_NBLK,),
        in_specs=[
            pl.BlockSpec((_G, _DIM), lambda i: (0, 0)),
            pl.BlockSpec((_G, _DIM), lambda i: (0, 0)),
            pl.BlockSpec((_G, _LAT), lambda i: (0, 0)),
            pl.BlockSpec((1, 1, _ROWS), lambda i: (i, 0, 0)),
            pl.BlockSpec((_LAT, _DIM), lambda i: (0, 0)),
            pl.BlockSpec((1, _DIM), lambda i: (0, 0)),
            pl.BlockSpec((_DIM, _DIM), lambda i: (0, 0)),
            pl.BlockSpec((1, _DIM), lambda i: (0, 0)),
        ],
        out_specs=[
            pl.BlockSpec((_ROWS, _DIM), lambda i: (i, 0)),
            pl.BlockSpec((_G, _LAT), lambda i: (0, 0)),
            pl.BlockSpec((_G, _LAT), lambda i: (0, 0)),
        ],
        out_shape=[
            jax.ShapeDtypeStruct((_N, _DIM), jnp.float32),
            jax.ShapeDtypeStruct((_G, _LAT), jnp.float32),
            jax.ShapeDtypeStruct((_G, _LAT), jnp.float32),
        ],
    )(sums, cnt, eps, batch3, w1, b1.reshape(1, _DIM), w2, b2.reshape(1, _DIM))


def kernel(x, edge_index, edge_attr, batch, params):
    src = edge_index[0]
    dst = edge_index[1]
    eps = jax.random.normal(jax.random.key(42), (_G, _LAT), jnp.float32)
    zeros_h = jnp.zeros((_NP, _DIM), jnp.float32)
    batch3 = batch.reshape(_NBLK, 1, _ROWS)

    ea_pad = jnp.pad(edge_attr, ((0, 0), (0, _DIM - _DE)))
    eagg = _sc_agg_ea(ea_pad, dst, dst, zeros_h)
    h = x
    for l in range(3):
        hagg = _sc_agg_h(h, src, dst, zeros_h)
        h = _layer_call(h, hagg, eagg, params['W_self'][l],
                        params['W_nbr'][l], jnp.pad(params['W_edge'][l], ((0, _DIM - _DE), (0, 0))),
                        params['b'][l], act=(l < 2))
    sums, cnt = _readout_call(h, batch3)
    recon, mu, logvar = _decode_call(
        sums, cnt, eps, batch3, params['mlp_W1'], params['mlp_b1'],
        params['mlp_W2'], params['mlp_b2'])
    return (recon, mu, logvar)


# speedup vs baseline: 8.8005x; 1.2139x over previous
import functools

import jax
import jax.numpy as jnp
from jax import lax
from jax.experimental import pallas as pl
from jax.experimental.pallas import tpu as pltpu
from jax.experimental.pallas import tpu_sc as plsc

_N = 10000
_E = 320000
_DIM = 128
_DE = 16
_G = 64
_LAT = 64

_NC = 2
_NS = 16
_NW = _NC * _NS
_EPW = _E // _NW
_CH = 128
_NFULL = _EPW // _CH
_REM = _EPW - _NFULL * _CH
_NP = 10240
_RPT = _NP // _NS

_ROWS = 1000
_NBLK = _N // _ROWS


def _make_sc_agg(depth, gather):
    mesh = plsc.VectorSubcoreMesh(
        core_axis_name="c", subcore_axis_name="s",
        num_cores=_NC, num_subcores=_NS)

    npair = _NFULL // 2

    @functools.partial(
        pl.kernel,
        out_type=jax.ShapeDtypeStruct((_NC, _NP, depth), jnp.float32),
        mesh=mesh,
        scratch_types=[
            pltpu.VMEM((_CH,), jnp.int32),
            pltpu.VMEM((_CH,), jnp.int32),
            pltpu.VMEM((_CH, depth), jnp.float32),
            pltpu.VMEM((_CH,), jnp.int32),
            pltpu.VMEM((_CH,), jnp.int32),
            pltpu.VMEM((_CH, depth), jnp.float32),
            pltpu.VMEM((_REM,), jnp.int32),
            pltpu.VMEM((_REM,), jnp.int32),
            pltpu.VMEM((_REM, depth), jnp.float32),
            pltpu.VMEM((_NFULL * _CH,), jnp.int32),
            pltpu.VMEM_SHARED((_NP, depth), jnp.float32),
            pltpu.SemaphoreType.DMA,
            pltpu.SemaphoreType.DMA,
            pltpu.SemaphoreType.DMA,
            pltpu.SemaphoreType.DMA,
        ],
    )
    def sc_agg(rows_hbm, src_hbm, dst_hbm, zeros_hbm, out_hbm,
               src0, dst0, buf0, src1, dst1, buf1,
               srcr_v, dstr_v, bufr_v, src_all, acc_sh,
               sem0, sem1, dsem0, dsem1):
        cid = lax.axis_index("c")
        sid = lax.axis_index("s")
        zoff = pl.multiple_of(sid * _RPT, 8)
        pltpu.sync_copy(zeros_hbm.at[pl.ds(zoff, _RPT)],
                        acc_sh.at[pl.ds(zoff, _RPT)])
        plsc.subcore_barrier()

        base = (cid * _NS + sid) * _EPW

        if gather:
            pltpu.sync_copy(src_hbm.at[pl.ds(base, _NFULL * _CH)], src_all)

        def load_and_start(c, dstv, bufv, semv, dsemv):
            off = pl.multiple_of(base + c * _CH, 8)
            pltpu.async_copy(dst_hbm.at[pl.ds(off, _CH)], dstv, dsemv)
            if gather:
                coff = pl.multiple_of(c * _CH, 8)
                pltpu.async_copy(
                    rows_hbm.at[src_all.at[pl.ds(coff, _CH)]], bufv, semv)
            else:
                pltpu.async_copy(rows_hbm.at[pl.ds(off, _CH)], bufv, semv)

        def wait_and_scatter(bufv, dstv, semv, dsemv):
            pltpu.make_async_copy(
                rows_hbm.at[pl.ds(0, _CH)], bufv, semv).wait()
            pltpu.make_async_copy(
                dst_hbm.at[pl.ds(0, _CH)], dstv, dsemv).wait()
            pltpu.sync_copy(bufv, acc_sh.at[dstv], add=True)

        load_and_start(0, dst0, buf0, sem0, dsem0)

        def body(i, carry):
            load_and_start(2 * i + 1, dst1, buf1, sem1, dsem1)
            wait_and_scatter(buf0, dst0, sem0, dsem0)

            @pl.when(i < npair - 1)
            def _():
                load_and_start(2 * i + 2, dst0, buf0, sem0, dsem0)

            wait_and_scatter(buf1, dst1, sem1, dsem1)
            return carry

        lax.fori_loop(0, npair, body, 0)

        offr = pl.multiple_of(base + _NFULL * _CH, 8)
        pltpu.sync_copy(dst_hbm.at[pl.ds(offr, _REM)], dstr_v)
        if gather:
            pltpu.sync_copy(src_hbm.at[pl.ds(offr, _REM)], srcr_v)
            pltpu.async_copy(rows_hbm.at[srcr_v], bufr_v, sem0).wait()
        else:
            pltpu.sync_copy(rows_hbm.at[pl.ds(offr, _REM)], bufr_v)
        pltpu.sync_copy(bufr_v, acc_sh.at[dstr_v], add=True)

        plsc.subcore_barrier()
        pltpu.sync_copy(acc_sh.at[pl.ds(zoff, _RPT)],
                        out_hbm.at[cid, pl.ds(zoff, _RPT)])

    return sc_agg


@functools.lru_cache(maxsize=None)
def _sc_agg(depth, gather):
    return _make_sc_agg(depth, gather)


def _sc_agg_h(rows, src, dst, zeros):
    return _sc_agg(_DIM, True)(rows, src, dst, zeros)


def _sc_agg_ea(rows, src, dst, zeros):
    return _sc_agg(_DIM, False)(rows, src, dst, zeros)


def _layer_body(h_ref, hagg_ref, eagg_ref, ws_ref, wn_ref, we_ref, b_ref,
                o_ref, *, act):
    f32 = jnp.float32
    acc = jnp.dot(h_ref[...], ws_ref[...], preferred_element_type=f32)
    acc += jnp.dot(hagg_ref[0] + hagg_ref[1], wn_ref[...],
                   preferred_element_type=f32)
    acc += jnp.dot(eagg_ref[0] + eagg_ref[1], we_ref[...],
                   preferred_element_type=f32)
    acc += b_ref[...]
    if act:
        acc = jnp.maximum(acc, 0.0)
    o_ref[...] = acc


def _layer_call(h, hagg, eagg, ws, wn, we, b, act):
    return pl.pallas_call(
        functools.partial(_layer_body, act=act),
        grid=(_NBLK,),
        in_specs=[
            pl.BlockSpec((_ROWS, _DIM), lambda i: (i, 0)),
            pl.BlockSpec((_NC, _ROWS, _DIM), lambda i: (0, i, 0)),
            pl.BlockSpec((_NC, _ROWS, _DIM), lambda i: (0, i, 0)),
            pl.BlockSpec((_DIM, _DIM), lambda i: (0, 0)),
            pl.BlockSpec((_DIM, _DIM), lambda i: (0, 0)),
            pl.BlockSpec((_DIM, _DIM), lambda i: (0, 0)),
            pl.BlockSpec((1, _DIM), lambda i: (0, 0)),
        ],
        out_specs=pl.BlockSpec((_ROWS, _DIM), lambda i: (i, 0)),
        out_shape=jax.ShapeDtypeStruct((_N, _DIM), jnp.float32),
    )(h, hagg, eagg, ws, wn, we, b.reshape(1, _DIM))


def _onehot(batch_ref):
    bvals = batch_ref[0, 0, :]
    return (bvals[:, None] == lax.broadcasted_iota(
        jnp.int32, (1, _G), 1)).astype(jnp.float32)


def _readout_body(h_ref, batch_ref, sums_ref, cnt_ref):
    i = pl.program_id(0)
    oh = _onehot(batch_ref)
    s = lax.dot_general(oh, h_ref[...], (((0,), (0,)), ((), ())),
                        preferred_element_type=jnp.float32)
    c = jnp.broadcast_to(jnp.sum(oh, axis=0)[:, None], (_G, _DIM))

    @pl.when(i == 0)
    def _():
        sums_ref[...] = s
        cnt_ref[...] = c

    @pl.when(i > 0)
    def _():
        sums_ref[...] += s
        cnt_ref[...] += c


def _readout_call(h, batch3):
    return pl.pallas_call(
        _readout_body,
        grid=(_NBLK,),
        in_specs=[
            pl.BlockSpec((_ROWS, _DIM), lambda i: (i, 0)),
            pl.BlockSpec((1, 1, _ROWS), lambda i: (i, 0, 0)),
        ],
        out_specs=[
            pl.BlockSpec((_G, _DIM), lambda i: (0, 0)),
            pl.BlockSpec((_G, _DIM), lambda i: (0, 0)),
        ],
        out_shape=[
            jax.ShapeDtypeStruct((_G, _DIM), jnp.float32),
            jax.ShapeDtypeStruct((_G, _DIM), jnp.float32),
        ],
    )(h, batch3)


def _decode_body(sums_ref, cnt_ref, eps_ref, batch_ref, w1_ref, b1_ref,
                 w2_ref, b2_ref, recon_ref, mu_ref, logvar_ref):
    f32 = jnp.float32
    g = sums_ref[...] / jnp.maximum(cnt_ref[...], 1.0)
    mu = g[:, :_LAT]
    logvar = g[:, _LAT:]
    mu_ref[...] = mu
    logvar_ref[...] = logvar
    z = mu + eps_ref[...] * jnp.exp(0.5 * logvar)
    oh = _onehot(batch_ref)
    z_exp = jnp.dot(oh, z, preferred_element_type=f32)
    hmid = jnp.maximum(
        jnp.dot(z_exp, w1_ref[...], preferred_element_type=f32) + b1_ref[...],
        0.0)
    recon_ref[...] = (
        jnp.dot(hmid, w2_ref[...], preferred_element_type=f32) + b2_ref[...])


def _decode_call(sums, cnt, eps, batch3, w1, b1, w2, b2):
    return pl.pallas_call(
        _decode_body,
        grid=(_NBLK,),
        in_specs=[
            pl.BlockSpec((_G, _DIM), lambda i: (0, 0)),
            pl.BlockSpec((_G, _DIM), lambda i: (0, 0)),
            pl.BlockSpec((_G, _LAT), lambda i: (0, 0)),
            pl.BlockSpec((1, 1, _ROWS), lambda i: (i, 0, 0)),
            pl.BlockSpec((_LAT, _DIM), lambda i: (0, 0)),
            pl.BlockSpec((1, _DIM), lambda i: (0, 0)),
            pl.BlockSpec((_DIM, _DIM), lambda i: (0, 0)),
            pl.BlockSpec((1, _DIM), lambda i: (0, 0)),
        ],
        out_specs=[
            pl.BlockSpec((_ROWS, _DIM), lambda i: (i, 0)),
            pl.BlockSpec((_G, _LAT), lambda i: (0, 0)),
            pl.BlockSpec((_G, _LAT), lambda i: (0, 0)),
        ],
        out_shape=[
            jax.ShapeDtypeStruct((_N, _DIM), jnp.float32),
            jax.ShapeDtypeStruct((_G, _LAT), jnp.float32),
            jax.ShapeDtypeStruct((_G, _LAT), jnp.float32),
        ],
    )(sums, cnt, eps, batch3, w1, b1.reshape(1, _DIM), w2, b2.reshape(1, _DIM))


def kernel(x, edge_index, edge_attr, batch, params):
    src = edge_index[0]
    dst = edge_index[1]
    eps = jax.random.normal(jax.random.key(42), (_G, _LAT), jnp.float32)
    zeros_h = jnp.zeros((_NP, _DIM), jnp.float32)
    batch3 = batch.reshape(_NBLK, 1, _ROWS)

    ea_pad = jnp.pad(edge_attr, ((0, 0), (0, _DIM - _DE)))
    eagg = _sc_agg_ea(ea_pad, dst, dst, zeros_h)
    h = x
    for l in range(3):
        hagg = _sc_agg_h(h, src, dst, zeros_h)
        h = _layer_call(h, hagg, eagg, params['W_self'][l],
                        params['W_nbr'][l], jnp.pad(params['W_edge'][l], ((0, _DIM - _DE), (0, 0))),
                        params['b'][l], act=(l < 2))
    sums, cnt = _readout_call(h, batch3)
    recon, mu, logvar = _decode_call(
        sums, cnt, eps, batch3, params['mlp_W1'], params['mlp_b1'],
        params['mlp_W2'], params['mlp_b2'])
    return (recon, mu, logvar)


# speedup vs baseline: 9.4142x; 1.0697x over previous
import functools

import jax
import jax.numpy as jnp
from jax import lax
from jax.experimental import pallas as pl
from jax.experimental.pallas import tpu as pltpu
from jax.experimental.pallas import tpu_sc as plsc

_N = 10000
_E = 320000
_DIM = 128
_DE = 16
_G = 64
_LAT = 64

_NC = 2
_NS = 16
_NW = _NC * _NS
_EPW = _E // _NW
_CH = 80
_NFULL = _EPW // _CH
_REM = _EPW - _NFULL * _CH
_NP = 10240
_RPT = _NP // _NS

_ROWS = 1000
_NBLK = _N // _ROWS


def _make_sc_agg(depth, gather):
    mesh = plsc.VectorSubcoreMesh(
        core_axis_name="c", subcore_axis_name="s",
        num_cores=_NC, num_subcores=_NS)

    nbuf = 3
    nring = _NFULL // nbuf
    ntail = _NFULL - nbuf * nring

    @functools.partial(
        pl.kernel,
        out_type=jax.ShapeDtypeStruct((_NC, _NP, depth), jnp.float32),
        mesh=mesh,
        scratch_types=[
            pltpu.VMEM((_CH,), jnp.int32),
            pltpu.VMEM((_CH,), jnp.int32),
            pltpu.VMEM((_CH,), jnp.int32),
            pltpu.VMEM((_CH, depth), jnp.float32),
            pltpu.VMEM((_CH, depth), jnp.float32),
            pltpu.VMEM((_CH, depth), jnp.float32),
            pltpu.VMEM((max(_REM, 8),), jnp.int32),
            pltpu.VMEM((max(_REM, 8),), jnp.int32),
            pltpu.VMEM((max(_REM, 8), depth), jnp.float32),
            pltpu.VMEM((_NFULL * _CH,), jnp.int32),
            pltpu.VMEM_SHARED((_NP, depth), jnp.float32),
            pltpu.SemaphoreType.DMA,
            pltpu.SemaphoreType.DMA,
            pltpu.SemaphoreType.DMA,
            pltpu.SemaphoreType.DMA,
            pltpu.SemaphoreType.DMA,
            pltpu.SemaphoreType.DMA,
        ],
    )
    def sc_agg(rows_hbm, src_hbm, dst_hbm, zeros_hbm, out_hbm,
               dst_a, dst_b, dst_c, buf_a, buf_b, buf_c,
               srcr_v, dstr_v, bufr_v, src_all, acc_sh,
               sem_a, sem_b, sem_c, dsem_a, dsem_b, dsem_c):
        dsts = (dst_a, dst_b, dst_c)
        bufs = (buf_a, buf_b, buf_c)
        sems = (sem_a, sem_b, sem_c)
        dsems = (dsem_a, dsem_b, dsem_c)
        cid = lax.axis_index("c")
        sid = lax.axis_index("s")
        zoff = pl.multiple_of(sid * _RPT, 8)
        pltpu.sync_copy(zeros_hbm.at[pl.ds(zoff, _RPT)],
                        acc_sh.at[pl.ds(zoff, _RPT)])
        plsc.subcore_barrier()

        base = (cid * _NS + sid) * _EPW

        if gather:
            pltpu.sync_copy(src_hbm.at[pl.ds(base, _NFULL * _CH)], src_all)

        def load_and_start(c, j):
            off = pl.multiple_of(base + c * _CH, 8)
            pltpu.async_copy(dst_hbm.at[pl.ds(off, _CH)], dsts[j], dsems[j])
            if gather:
                coff = pl.multiple_of(c * _CH, 8)
                pltpu.async_copy(
                    rows_hbm.at[src_all.at[pl.ds(coff, _CH)]],
                    bufs[j], sems[j])
            else:
                pltpu.async_copy(rows_hbm.at[pl.ds(off, _CH)],
                                 bufs[j], sems[j])

        def wait_and_scatter(j):
            pltpu.make_async_copy(
                rows_hbm.at[pl.ds(0, _CH)], bufs[j], sems[j]).wait()
            pltpu.make_async_copy(
                dst_hbm.at[pl.ds(0, _CH)], dsts[j], dsems[j]).wait()
            pltpu.sync_copy(bufs[j], acc_sh.at[dsts[j]], add=True)

        for j in range(nbuf - 1):
            load_and_start(j, j)

        def body(i, carry):
            for j in range(nbuf):
                c = nbuf * i + j

                @pl.when(c + nbuf - 1 < _NFULL)
                def _():
                    load_and_start(c + nbuf - 1, (j + nbuf - 1) % nbuf)

                wait_and_scatter(j)
            return carry

        lax.fori_loop(0, nring, body, 0)
        for t in range(ntail):
            wait_and_scatter(t)

        if _REM:
            offr = pl.multiple_of(base + _NFULL * _CH, 8)
            pltpu.sync_copy(dst_hbm.at[pl.ds(offr, _REM)], dstr_v)
            if gather:
                pltpu.sync_copy(src_hbm.at[pl.ds(offr, _REM)], srcr_v)
                pltpu.async_copy(rows_hbm.at[srcr_v], bufr_v, sems[0]).wait()
            else:
                pltpu.sync_copy(rows_hbm.at[pl.ds(offr, _REM)], bufr_v)
            pltpu.sync_copy(bufr_v, acc_sh.at[dstr_v], add=True)

        plsc.subcore_barrier()
        pltpu.sync_copy(acc_sh.at[pl.ds(zoff, _RPT)],
                        out_hbm.at[cid, pl.ds(zoff, _RPT)])

    return sc_agg


@functools.lru_cache(maxsize=None)
def _sc_agg(depth, gather):
    return _make_sc_agg(depth, gather)


def _sc_agg_h(rows, src, dst, zeros):
    return _sc_agg(_DIM, True)(rows, src, dst, zeros)


def _sc_agg_ea(rows, src, dst, zeros):
    return _sc_agg(_DIM, False)(rows, src, dst, zeros)


def _layer_body(h_ref, hagg_ref, eagg_ref, ws_ref, wn_ref, we_ref, b_ref,
                o_ref, *, act):
    f32 = jnp.float32
    acc = jnp.dot(h_ref[...], ws_ref[...], preferred_element_type=f32)
    acc += jnp.dot(hagg_ref[0] + hagg_ref[1], wn_ref[...],
                   preferred_element_type=f32)
    acc += jnp.dot(eagg_ref[0] + eagg_ref[1], we_ref[...],
                   preferred_element_type=f32)
    acc += b_ref[...]
    if act:
        acc = jnp.maximum(acc, 0.0)
    o_ref[...] = acc


def _layer_call(h, hagg, eagg, ws, wn, we, b, act):
    return pl.pallas_call(
        functools.partial(_layer_body, act=act),
        grid=(_NBLK,),
        in_specs=[
            pl.BlockSpec((_ROWS, _DIM), lambda i: (i, 0)),
            pl.BlockSpec((_NC, _ROWS, _DIM), lambda i: (0, i, 0)),
            pl.BlockSpec((_NC, _ROWS, _DIM), lambda i: (0, i, 0)),
            pl.BlockSpec((_DIM, _DIM), lambda i: (0, 0)),
            pl.BlockSpec((_DIM, _DIM), lambda i: (0, 0)),
            pl.BlockSpec((_DIM, _DIM), lambda i: (0, 0)),
            pl.BlockSpec((1, _DIM), lambda i: (0, 0)),
        ],
        out_specs=pl.BlockSpec((_ROWS, _DIM), lambda i: (i, 0)),
        out_shape=jax.ShapeDtypeStruct((_N, _DIM), jnp.float32),
    )(h, hagg, eagg, ws, wn, we, b.reshape(1, _DIM))


def _onehot(batch_ref):
    bvals = batch_ref[0, 0, :]
    return (bvals[:, None] == lax.broadcasted_iota(
        jnp.int32, (1, _G), 1)).astype(jnp.float32)


def _readout_body(h_ref, batch_ref, sums_ref, cnt_ref):
    i = pl.program_id(0)
    oh = _onehot(batch_ref)
    s = lax.dot_general(oh, h_ref[...], (((0,), (0,)), ((), ())),
                        preferred_element_type=jnp.float32)
    c = jnp.broadcast_to(jnp.sum(oh, axis=0)[:, None], (_G, _DIM))

    @pl.when(i == 0)
    def _():
        sums_ref[...] = s
        cnt_ref[...] = c

    @pl.when(i > 0)
    def _():
        sums_ref[...] += s
        cnt_ref[...] += c


def _readout_call(h, batch3):
    return pl.pallas_call(
        _readout_body,
        grid=(_NBLK,),
        in_specs=[
            pl.BlockSpec((_ROWS, _DIM), lambda i: (i, 0)),
            pl.BlockSpec((1, 1, _ROWS), lambda i: (i, 0, 0)),
        ],
        out_specs=[
            pl.BlockSpec((_G, _DIM), lambda i: (0, 0)),
            pl.BlockSpec((_G, _DIM), lambda i: (0, 0)),
        ],
        out_shape=[
            jax.ShapeDtypeStruct((_G, _DIM), jnp.float32),
            jax.ShapeDtypeStruct((_G, _DIM), jnp.float32),
        ],
    )(h, batch3)


def _decode_body(sums_ref, cnt_ref, eps_ref, batch_ref, w1_ref, b1_ref,
                 w2_ref, b2_ref, recon_ref, mu_ref, logvar_ref):
    f32 = jnp.float32
    g = sums_ref[...] / jnp.maximum(cnt_ref[...], 1.0)
    mu = g[:, :_LAT]
    logvar = g[:, _LAT:]
    mu_ref[...] = mu
    logvar_ref[...] = logvar
    z = mu + eps_ref[...] * jnp.exp(0.5 * logvar)
    oh = _onehot(batch_ref)
    z_exp = jnp.dot(oh, z, preferred_element_type=f32)
    hmid = jnp.maximum(
        jnp.dot(z_exp, w1_ref[...], preferred_element_type=f32) + b1_ref[...],
        0.0)
    recon_ref[...] = (
        jnp.dot(hmid, w2_ref[...], preferred_element_type=f32) + b2_ref[...])


def _decode_call(sums, cnt, eps, batch3, w1, b1, w2, b2):
    return pl.pallas_call(
        _decode_body,
        grid=(_NBLK,),
        in_specs=[
            pl.BlockSpec((_G, _DIM), lambda i: (0, 0)),
            pl.BlockSpec((_G, _DIM), lambda i: (0, 0)),
            pl.BlockSpec((_G, _LAT), lambda i: (0, 0)),
            pl.BlockSpec((1, 1, _ROWS), lambda i: (i, 0, 0)),
            pl.BlockSpec((_LAT, _DIM), lambda i: (0, 0)),
            pl.BlockSpec((1, _DIM), lambda i: (0, 0)),
            pl.BlockSpec((_DIM, _DIM), lambda i: (0, 0)),
            pl.BlockSpec((1, _DIM), lambda i: (0, 0)),
        ],
        out_specs=[
            pl.BlockSpec((_ROWS, _DIM), lambda i: (i, 0)),
            pl.BlockSpec((_G, _LAT), lambda i: (0, 0)),
            pl.BlockSpec((_G, _LAT), lambda i: (0, 0)),
        ],
        out_shape=[
            jax.ShapeDtypeStruct((_N, _DIM), jnp.float32),
            jax.ShapeDtypeStruct((_G, _LAT), jnp.float32),
            jax.ShapeDtypeStruct((_G, _LAT), jnp.float32),
        ],
    )(sums, cnt, eps, batch3, w1, b1.reshape(1, _DIM), w2, b2.reshape(1, _DIM))


def kernel(x, edge_index, edge_attr, batch, params):
    src = edge_index[0]
    dst = edge_index[1]
    eps = jax.random.normal(jax.random.key(42), (_G, _LAT), jnp.float32)
    zeros_h = jnp.zeros((_NP, _DIM), jnp.float32)
    batch3 = batch.reshape(_NBLK, 1, _ROWS)

    ea_pad = jnp.pad(edge_attr, ((0, 0), (0, _DIM - _DE)))
    eagg = _sc_agg_ea(ea_pad, dst, dst, zeros_h)
    h = x
    for l in range(3):
        hagg = _sc_agg_h(h, src, dst, zeros_h)
        h = _layer_call(h, hagg, eagg, params['W_self'][l],
                        params['W_nbr'][l], jnp.pad(params['W_edge'][l], ((0, _DIM - _DE), (0, 0))),
                        params['b'][l], act=(l < 2))
    sums, cnt = _readout_call(h, batch3)
    recon, mu, logvar = _decode_call(
        sums, cnt, eps, batch3, params['mlp_W1'], params['mlp_b1'],
        params['mlp_W2'], params['mlp_b2'])
    return (recon, mu, logvar)


# speedup vs baseline: 10.4340x; 1.1083x over previous
import functools

import jax
import jax.numpy as jnp
from jax import lax
from jax.experimental import pallas as pl
from jax.experimental.pallas import tpu as pltpu
from jax.experimental.pallas import tpu_sc as plsc

_N = 10000
_E = 320000
_DIM = 128
_DE = 16
_G = 64
_LAT = 64

_NC = 2
_NS = 16
_NW = _NC * _NS
_EPW = _E // _NW
_CH = 80
_NFULL = _EPW // _CH
_REM = _EPW - _NFULL * _CH
_NP = 10240
_RPT = _NP // _NS

_ROWS = 1000
_NBLK = _N // _ROWS


def _make_sc_agg(depth, gather):
    mesh = plsc.VectorSubcoreMesh(
        core_axis_name="c", subcore_axis_name="s",
        num_cores=_NC, num_subcores=_NS)

    nbuf = 3
    nring = _NFULL // nbuf
    ntail = _NFULL - nbuf * nring

    @functools.partial(
        pl.kernel,
        out_type=jax.ShapeDtypeStruct((_NC, _NP, depth), jnp.float32),
        mesh=mesh,
        scratch_types=[
            pltpu.VMEM((_CH,), jnp.int32),
            pltpu.VMEM((_CH,), jnp.int32),
            pltpu.VMEM((_CH,), jnp.int32),
            pltpu.VMEM((_CH, depth), jnp.float32),
            pltpu.VMEM((_CH, depth), jnp.float32),
            pltpu.VMEM((_CH, depth), jnp.float32),
            pltpu.VMEM((max(_REM, 8),), jnp.int32),
            pltpu.VMEM((max(_REM, 8),), jnp.int32),
            pltpu.VMEM((max(_REM, 8), depth), jnp.float32),
            pltpu.VMEM((_NFULL * _CH,), jnp.int32),
            pltpu.VMEM_SHARED((_NP, depth), jnp.float32),
            pltpu.SemaphoreType.DMA,
            pltpu.SemaphoreType.DMA,
            pltpu.SemaphoreType.DMA,
            pltpu.SemaphoreType.DMA,
            pltpu.SemaphoreType.DMA,
            pltpu.SemaphoreType.DMA,
        ],
    )
    def sc_agg(rows_hbm, src_hbm, dst_hbm, zeros_hbm, out_hbm,
               dst_a, dst_b, dst_c, buf_a, buf_b, buf_c,
               srcr_v, dstr_v, bufr_v, src_all, acc_sh,
               sem_a, sem_b, sem_c, dsem_a, dsem_b, dsem_c):
        dsts = (dst_a, dst_b, dst_c)
        bufs = (buf_a, buf_b, buf_c)
        sems = (sem_a, sem_b, sem_c)
        dsems = (dsem_a, dsem_b, dsem_c)
        cid = lax.axis_index("c")
        sid = lax.axis_index("s")
        zoff = pl.multiple_of(sid * _RPT, 8)
        pltpu.sync_copy(zeros_hbm.at[pl.ds(zoff, _RPT)],
                        acc_sh.at[pl.ds(zoff, _RPT)])
        plsc.subcore_barrier()

        base = (cid * _NS + sid) * _EPW

        if gather:
            pltpu.sync_copy(src_hbm.at[pl.ds(base, _NFULL * _CH)], src_all)

        def load_and_start(c, j):
            off = pl.multiple_of(base + c * _CH, 8)
            pltpu.async_copy(dst_hbm.at[pl.ds(off, _CH)], dsts[j], dsems[j])
            if gather:
                coff = pl.multiple_of(c * _CH, 8)
                pltpu.async_copy(
                    rows_hbm.at[src_all.at[pl.ds(coff, _CH)]],
                    bufs[j], sems[j])
            else:
                pltpu.async_copy(rows_hbm.at[pl.ds(off, _CH)],
                                 bufs[j], sems[j])

        def wait_and_scatter(j):
            pltpu.make_async_copy(
                rows_hbm.at[pl.ds(0, _CH)], bufs[j], sems[j]).wait()
            pltpu.make_async_copy(
                dst_hbm.at[pl.ds(0, _CH)], dsts[j], dsems[j]).wait()
            pltpu.sync_copy(bufs[j], acc_sh.at[dsts[j]], add=True)

        for j in range(nbuf - 1):
            load_and_start(j, j)

        def body(i, carry):
            for j in range(nbuf):
                c = nbuf * i + j

                @pl.when(c + nbuf - 1 < _NFULL)
                def _():
                    load_and_start(c + nbuf - 1, (j + nbuf - 1) % nbuf)

                wait_and_scatter(j)
            return carry

        lax.fori_loop(0, nring, body, 0)
        for t in range(ntail):
            wait_and_scatter(t)

        if _REM:
            offr = pl.multiple_of(base + _NFULL * _CH, 8)
            pltpu.sync_copy(dst_hbm.at[pl.ds(offr, _REM)], dstr_v)
            if gather:
                pltpu.sync_copy(src_hbm.at[pl.ds(offr, _REM)], srcr_v)
                pltpu.async_copy(rows_hbm.at[srcr_v], bufr_v, sems[0]).wait()
            else:
                pltpu.sync_copy(rows_hbm.at[pl.ds(offr, _REM)], bufr_v)
            pltpu.sync_copy(bufr_v, acc_sh.at[dstr_v], add=True)

        plsc.subcore_barrier()
        pltpu.sync_copy(acc_sh.at[pl.ds(zoff, _RPT)],
                        out_hbm.at[cid, pl.ds(zoff, _RPT)])

    return sc_agg


_PCH = 64
_PEPW = 9984
_PNF = _PEPW // _PCH
_PLEFT_CHUNKS = (_E - _NW * _PEPW) // _PCH


def _make_sc_agg_packed():
    mesh = plsc.VectorSubcoreMesh(
        core_axis_name="c", subcore_axis_name="s",
        num_cores=_NC, num_subcores=_NS)

    nbuf = 3
    nring = _PNF // nbuf
    ntail = _PNF - nbuf * nring

    @functools.partial(
        pl.kernel,
        out_type=jax.ShapeDtypeStruct((_NC, _NP, _DIM), jnp.float32),
        mesh=mesh,
        scratch_types=[
            pltpu.VMEM((_PCH,), jnp.int32),
            pltpu.VMEM((_PCH,), jnp.int32),
            pltpu.VMEM((_PCH,), jnp.int32),
            pltpu.VMEM((8, _DIM), jnp.float32),
            pltpu.VMEM((8, _DIM), jnp.float32),
            pltpu.VMEM((8, _DIM), jnp.float32),
            pltpu.VMEM((_PCH, _DIM), jnp.float32),
            pltpu.VMEM((_PCH, _DIM), jnp.float32),
            pltpu.VMEM((_PCH, _DIM), jnp.float32),
            pltpu.VMEM_SHARED((_NP, _DIM), jnp.float32),
            pltpu.SemaphoreType.DMA,
            pltpu.SemaphoreType.DMA,
            pltpu.SemaphoreType.DMA,
            pltpu.SemaphoreType.DMA,
            pltpu.SemaphoreType.DMA,
            pltpu.SemaphoreType.DMA,
        ],
    )
    def sc_agg_packed(rows_hbm, dst_hbm, zeros_hbm, out_hbm,
                      dst_a, dst_b, dst_c, pk_a, pk_b, pk_c,
                      buf_a, buf_b, buf_c, acc_sh,
                      psem_a, psem_b, psem_c, dsem_a, dsem_b, dsem_c):
        dsts = (dst_a, dst_b, dst_c)
        pks = (pk_a, pk_b, pk_c)
        bufs = (buf_a, buf_b, buf_c)
        psems = (psem_a, psem_b, psem_c)
        dsems = (dsem_a, dsem_b, dsem_c)
        cid = lax.axis_index("c")
        sid = lax.axis_index("s")
        wid = cid * _NS + sid
        zoff = pl.multiple_of(sid * _RPT, 8)
        pltpu.sync_copy(zeros_hbm.at[pl.ds(zoff, _RPT)],
                        acc_sh.at[pl.ds(zoff, _RPT)])
        for j in range(nbuf):
            pltpu.sync_copy(zeros_hbm.at[pl.ds(0, _PCH)], bufs[j])
        plsc.subcore_barrier()

        ebase = wid * _PEPW
        pbase = wid * (_PEPW // 8)

        def load_and_start(c, j):
            eoff = pl.multiple_of(ebase + c * _PCH, 8)
            poff = pl.multiple_of(pbase + c * (_PCH // 8), 8)
            pltpu.async_copy(dst_hbm.at[pl.ds(eoff, _PCH)], dsts[j], dsems[j])
            pltpu.async_copy(rows_hbm.at[pl.ds(poff, _PCH // 8)],
                             pks[j], psems[j])

        def wait_expand_scatter(j):
            pltpu.make_async_copy(
                rows_hbm.at[pl.ds(0, _PCH // 8)], pks[j], psems[j]).wait()
            for p in range(_PCH // 8):
                for q in range(8):
                    bufs[j][8 * p + q, pl.ds(0, _DE)] = (
                        pks[j][p, pl.ds(_DE * q, _DE)])
            pltpu.make_async_copy(
                dst_hbm.at[pl.ds(0, _PCH)], dsts[j], dsems[j]).wait()
            pltpu.sync_copy(bufs[j], acc_sh.at[dsts[j]], add=True)

        for j in range(nbuf - 1):
            load_and_start(j, j)

        def body(i, carry):
            for j in range(nbuf):
                c = nbuf * i + j

                @pl.when(c + nbuf - 1 < _PNF)
                def _():
                    load_and_start(c + nbuf - 1, (j + nbuf - 1) % nbuf)

                wait_expand_scatter(j)
            return carry

        lax.fori_loop(0, nring, body, 0)
        for t in range(ntail):
            wait_expand_scatter(t)

        @pl.when(wid < _PLEFT_CHUNKS)
        def _():
            eoff = pl.multiple_of(_NW * _PEPW + wid * _PCH, 8)
            poff = pl.multiple_of((_NW * _PEPW) // 8 + wid * (_PCH // 8), 8)
            pltpu.async_copy(dst_hbm.at[pl.ds(eoff, _PCH)], dsts[0], dsems[0])
            pltpu.async_copy(rows_hbm.at[pl.ds(poff, _PCH // 8)],
                             pks[0], psems[0])
            wait_expand_scatter(0)

        plsc.subcore_barrier()
        pltpu.sync_copy(acc_sh.at[pl.ds(zoff, _RPT)],
                        out_hbm.at[cid, pl.ds(zoff, _RPT)])

    return sc_agg_packed


@functools.lru_cache(maxsize=None)
def _sc_agg(depth, gather):
    return _make_sc_agg(depth, gather)


@functools.lru_cache(maxsize=None)
def _sc_agg_packed_cached():
    return _make_sc_agg_packed()


def _sc_agg_h(rows, src, dst, zeros):
    return _sc_agg(_DIM, True)(rows, src, dst, zeros)


def _sc_agg_ea(rows_packed, dst, zeros):
    return _sc_agg_packed_cached()(rows_packed, dst, zeros)


def _layer_body(h_ref, hagg_ref, eagg_ref, ws_ref, wn_ref, we_ref, b_ref,
                o_ref, *, act):
    f32 = jnp.float32
    acc = jnp.dot(h_ref[...], ws_ref[...], preferred_element_type=f32)
    acc += jnp.dot(hagg_ref[0] + hagg_ref[1], wn_ref[...],
                   preferred_element_type=f32)
    acc += jnp.dot(eagg_ref[0] + eagg_ref[1], we_ref[...],
                   preferred_element_type=f32)
    acc += b_ref[...]
    if act:
        acc = jnp.maximum(acc, 0.0)
    o_ref[...] = acc


def _layer_call(h, hagg, eagg, ws, wn, we, b, act):
    return pl.pallas_call(
        functools.partial(_layer_body, act=act),
        grid=(_NBLK,),
        in_specs=[
            pl.BlockSpec((_ROWS, _DIM), lambda i: (i, 0)),
            pl.BlockSpec((_NC, _ROWS, _DIM), lambda i: (0, i, 0)),
            pl.BlockSpec((_NC, _ROWS, _DIM), lambda i: (0, i, 0)),
            pl.BlockSpec((_DIM, _DIM), lambda i: (0, 0)),
            pl.BlockSpec((_DIM, _DIM), lambda i: (0, 0)),
            pl.BlockSpec((_DIM, _DIM), lambda i: (0, 0)),
            pl.BlockSpec((1, _DIM), lambda i: (0, 0)),
        ],
        out_specs=pl.BlockSpec((_ROWS, _DIM), lambda i: (i, 0)),
        out_shape=jax.ShapeDtypeStruct((_N, _DIM), jnp.float32),
    )(h, hagg, eagg, ws, wn, we, b.reshape(1, _DIM))


def _onehot(batch_ref):
    bvals = batch_ref[0, 0, :]
    return (bvals[:, None] == lax.broadcasted_iota(
        jnp.int32, (1, _G), 1)).astype(jnp.float32)


def _readout_body(h_ref, batch_ref, sums_ref, cnt_ref):
    i = pl.program_id(0)
    oh = _onehot(batch_ref)
    s = lax.dot_general(oh, h_ref[...], (((0,), (0,)), ((), ())),
                        preferred_element_type=jnp.float32)
    c = jnp.broadcast_to(jnp.sum(oh, axis=0)[:, None], (_G, _DIM))

    @pl.when(i == 0)
    def _():
        sums_ref[...] = s
        cnt_ref[...] = c

    @pl.when(i > 0)
    def _():
        sums_ref[...] += s
        cnt_ref[...] += c


def _readout_call(h, batch3):
    return pl.pallas_call(
        _readout_body,
        grid=(_NBLK,),
        in_specs=[
            pl.BlockSpec((_ROWS, _DIM), lambda i: (i, 0)),
            pl.BlockSpec((1, 1, _ROWS), lambda i: (i, 0, 0)),
        ],
        out_specs=[
            pl.BlockSpec((_G, _DIM), lambda i: (0, 0)),
            pl.BlockSpec((_G, _DIM), lambda i: (0, 0)),
        ],
        out_shape=[
            jax.ShapeDtypeStruct((_G, _DIM), jnp.float32),
            jax.ShapeDtypeStruct((_G, _DIM), jnp.float32),
        ],
    )(h, batch3)


def _decode_body(sums_ref, cnt_ref, eps_ref, batch_ref, w1_ref, b1_ref,
                 w2_ref, b2_ref, recon_ref, mu_ref, logvar_ref):
    f32 = jnp.float32
    g = sums_ref[...] / jnp.maximum(cnt_ref[...], 1.0)
    mu = g[:, :_LAT]
    logvar = g[:, _LAT:]
    mu_ref[...] = mu
    logvar_ref[...] = logvar
    z = mu + eps_ref[...] * jnp.exp(0.5 * logvar)
    oh = _onehot(batch_ref)
    z_exp = jnp.dot(oh, z, preferred_element_type=f32)
    hmid = jnp.maximum(
        jnp.dot(z_exp, w1_ref[...], preferred_element_type=f32) + b1_ref[...],
        0.0)
    recon_ref[...] = (
        jnp.dot(hmid, w2_ref[...], preferred_element_type=f32) + b2_ref[...])


def _decode_call(sums, cnt, eps, batch3, w1, b1, w2, b2):
    return pl.pallas_call(
        _decode_body,
        grid=(_NBLK,),
        in_specs=[
            pl.BlockSpec((_G, _DIM), lambda i: (0, 0)),
            pl.BlockSpec((_G, _DIM), lambda i: (0, 0)),
            pl.BlockSpec((_G, _LAT), lambda i: (0, 0)),
            pl.BlockSpec((1, 1, _ROWS), lambda i: (i, 0, 0)),
            pl.BlockSpec((_LAT, _DIM), lambda i: (0, 0)),
            pl.BlockSpec((1, _DIM), lambda i: (0, 0)),
            pl.BlockSpec((_DIM, _DIM), lambda i: (0, 0)),
            pl.BlockSpec((1, _DIM), lambda i: (0, 0)),
        ],
        out_specs=[
            pl.BlockSpec((_ROWS, _DIM), lambda i: (i, 0)),
            pl.BlockSpec((_G, _LAT), lambda i: (0, 0)),
            pl.BlockSpec((_G, _LAT), lambda i: (0, 0)),
        ],
        out_shape=[
            jax.ShapeDtypeStruct((_N, _DIM), jnp.float32),
            jax.ShapeDtypeStruct((_G, _LAT), jnp.float32),
            jax.ShapeDtypeStruct((_G, _LAT), jnp.float32),
        ],
    )(sums, cnt, eps, batch3, w1, b1.reshape(1, _DIM), w2, b2.reshape(1, _DIM))


def kernel(x, edge_index, edge_attr, batch, params):
    src = edge_index[0]
    dst = edge_index[1]
    eps = jax.random.normal(jax.random.key(42), (_G, _LAT), jnp.float32)
    zeros_h = jnp.zeros((_NP, _DIM), jnp.float32)
    batch3 = batch.reshape(_NBLK, 1, _ROWS)

    ea_packed = edge_attr.reshape(_E // 8, _DIM)
    eagg = _sc_agg_ea(ea_packed, dst, zeros_h)
    h = x
    for l in range(3):
        hagg = _sc_agg_h(h, src, dst, zeros_h)
        h = _layer_call(h, hagg, eagg, params['W_self'][l],
                        params['W_nbr'][l], jnp.pad(params['W_edge'][l], ((0, _DIM - _DE), (0, 0))),
                        params['b'][l], act=(l < 2))
    sums, cnt = _readout_call(h, batch3)
    recon, mu, logvar = _decode_call(
        sums, cnt, eps, batch3, params['mlp_W1'], params['mlp_b1'],
        params['mlp_W2'], params['mlp_b2'])
    return (recon, mu, logvar)
